# Initial kernel scaffold; baseline (speedup 1.0000x reference)
import jax
import jax.numpy as jnp
from jax.experimental import pallas as pl


def kernel(timestep, class_label, W1, b1, W2, b2, class_table):
    raise NotImplementedError("write your pallas kernel here")


# speedup vs baseline: 1.0635x; 1.0635x over previous
import functools

import jax
import jax.numpy as jnp
from jax import lax
from jax.experimental import pallas as pl
from jax.experimental.pallas import tpu as pltpu
from jax.experimental.pallas import tpu_sc as plsc

DIM = 1024
HALF = DIM // 2
COND = DIM * 4
BATCH = 4096
NUM_CLASSES = 1000
LOG_MAX_PERIOD = 9.210340371976184


_SC_INFO = plsc.get_sparse_core_info()
_NW = _SC_INFO.num_cores * _SC_INFO.num_subcores
_B_PER_W = BATCH // _NW
_CHUNK = 16
_NCHUNK = _B_PER_W // _CHUNK


@functools.partial(
    pl.kernel,
    mesh=plsc.VectorSubcoreMesh(core_axis_name="c", subcore_axis_name="s"),
    out_type=jax.ShapeDtypeStruct((BATCH, COND), jnp.float32),
    scratch_types=[
        pltpu.VMEM((_NCHUNK, _CHUNK), jnp.int32),
        pltpu.VMEM((_CHUNK, COND), jnp.float32),
        pltpu.SemaphoreType.DMA,
    ],
)
def _sc_gather(table_hbm, idx_hbm, out_hbm, idx_v, rows_v, sem):
    wid = lax.axis_index("s") * _SC_INFO.num_cores + lax.axis_index("c")
    base = wid * _B_PER_W
    pltpu.sync_copy(idx_hbm.at[wid], idx_v)
    for c in range(_NCHUNK):
        pltpu.async_copy(table_hbm.at[idx_v.at[c]], rows_v, sem).wait()
        pltpu.sync_copy(rows_v, out_hbm.at[pl.ds(base + c * _CHUNK, _CHUNK)])


_BB = 1024
_CB = 512
_NB = BATCH // _BB
_NC = COND // _CB


def _tc_body(t_ref, w1_ref, b1_ref, w2_ref, b2_ref, c_ref, out_ref, h_ref):
    @pl.when(pl.program_id(1) == 0)
    def _():
        t = t_ref[...]
        half_iota = lax.broadcasted_iota(jnp.int32, (1, HALF), 1).astype(jnp.float32)
        freqs = jnp.exp(half_iota * (-LOG_MAX_PERIOD / HALF))
        args = t * freqs
        emb = jnp.concatenate([jnp.cos(args), jnp.sin(args)], axis=1)
        h = jnp.dot(emb, w1_ref[...], preferred_element_type=jnp.float32)
        h = h + b1_ref[...]
        h_ref[...] = h * (1.0 / (1.0 + jnp.exp(-h)))

    acc = jnp.dot(h_ref[...], w2_ref[...], preferred_element_type=jnp.float32)
    out_ref[...] = acc + b2_ref[...] + c_ref[...]


def _tc_mlp(t2d, W1, b1, W2, b2, c_emb):
    return pl.pallas_call(
        _tc_body,
        grid=(_NB, _NC),
        in_specs=[
            pl.BlockSpec((_BB, 1), lambda i, j: (i, 0)),
            pl.BlockSpec((DIM, COND), lambda i, j: (0, 0)),
            pl.BlockSpec((1, COND), lambda i, j: (0, 0)),
            pl.BlockSpec((COND, _CB), lambda i, j: (0, j)),
            pl.BlockSpec((1, _CB), lambda i, j: (0, j)),
            pl.BlockSpec((_BB, _CB), lambda i, j: (i, j)),
        ],
        out_specs=pl.BlockSpec((_BB, _CB), lambda i, j: (i, j)),
        out_shape=jax.ShapeDtypeStruct((BATCH, COND), jnp.float32),
        scratch_shapes=[pltpu.VMEM((_BB, COND), jnp.float32)],
        compiler_params=pltpu.CompilerParams(
            vmem_limit_bytes=100 * 1024 * 1024,
        ),
    )(t2d, W1, b1, W2, b2, c_emb)


def kernel(timestep, class_label, W1, b1, W2, b2, class_table):
    c_emb = _sc_gather(
        class_table,
        class_label.astype(jnp.int32).reshape(_NW, _NCHUNK, _CHUNK),
    )
    return _tc_mlp(
        timestep.reshape(BATCH, 1),
        W1,
        b1.reshape(1, COND),
        W2,
        b2.reshape(1, COND),
        c_emb,
    )
